# baseline (speedup 1.0000x reference)
import jax
import jax.numpy as jnp
from jax.experimental import pallas as pl


def _mask_kernel(o_ref, *, x_len):
    col = jax.lax.broadcasted_iota(jnp.int32, o_ref.shape, 1)
    boundary_left = jnp.int32(0)
    boundary_right = jnp.int32(x_len)
    o_ref[...] = (col >= boundary_left) & (col < boundary_right)


def kernel(x, y):
    x_len = x.shape[1]
    del y
    block_rows = 512
    import functools
    return pl.pallas_call(
        functools.partial(_mask_kernel, x_len=x_len),
        out_shape=jax.ShapeDtypeStruct((x_len, x_len), jnp.bool_),
        grid=(x_len // block_rows,),
        out_specs=pl.BlockSpec((block_rows, x_len), lambda i: (i, 0)),
    )()
